# Initial kernel scaffold; baseline (speedup 1.0000x reference)
import jax
import jax.numpy as jnp
from jax.experimental import pallas as pl


def kernel(x, edge_index1, edge_index2, edge_index3, W_and, a_and):
    raise NotImplementedError("write your pallas kernel here")


# speedup vs baseline: 1.2008x; 1.2008x over previous
import functools

import jax
import jax.numpy as jnp
from jax import lax
from jax.experimental import pallas as pl
from jax.experimental.pallas import tpu as pltpu
from jax.experimental.pallas import tpu_sc as plsc

N = 10000
D = 256
E = 160000
NPAD = 10240
NB = 32
RPT = NPAD // NB
CAP = 80128
FLAT = 2 * NB * CAP
MAGIC = 52429
SENT = NPAD
EH = E // 2
EPW = EH // 16
NV = 5008 // 16
LBUF = 5280
DUMP0 = 5256
CH = 80

_f32 = jnp.float32
_i32 = jnp.int32

_sc_mesh = plsc.VectorSubcoreMesh(core_axis_name="c", subcore_axis_name="s")
_LANES = None


def _lanes():
    return lax.iota(_i32, 16)


def _al8(x):
    return pl.multiple_of(x, 8)


def _z16():
    return jnp.zeros((16,), _f32)


def _zi16():
    return jnp.zeros((16,), _i32)


def _make_bucket(companion):
    outs = [jax.ShapeDtypeStruct((FLAT,), _i32),
            jax.ShapeDtypeStruct((2 * NB * 16,), _i32)]
    scratch = [
        pltpu.VMEM((5008,), _i32),
        pltpu.VMEM((LBUF,), _i32),
        pltpu.VMEM((32,), _i32),
        pltpu.VMEM((512,), _i32),
        pltpu.VMEM((288,), _i32),
        pltpu.VMEM((2 * NB * 16 // 2,), _i32),
        pltpu.VMEM_SHARED((512,), _i32),
        pltpu.SemaphoreType.DMA,
    ]
    if companion:
        outs = [jax.ShapeDtypeStruct((FLAT,), _i32)] + outs
        scratch = [pltpu.VMEM((5008,), _i32),
                   pltpu.VMEM((LBUF,), _i32)] + scratch

    @functools.partial(pl.kernel, out_type=tuple(outs), mesh=_sc_mesh,
                       scratch_types=tuple(scratch))
    def bucket(*refs):
        if companion:
            (src_hbm, dst_hbm, bsrc_hbm, bdst_hbm, tcnt_hbm,
             sstage, sloc, dstage, dloc, pub, tbl, posb, totv, spm, sem) = refs
        else:
            (dst_hbm, bdst_hbm, tcnt_hbm,
             dstage, dloc, pub, tbl, posb, totv, spm, sem) = refs
        c = lax.axis_index("c")
        s = lax.axis_index("s")
        lanes = _lanes()
        base_e = c * EH + s * EPW

        pltpu.sync_copy(dst_hbm.at[pl.ds(_al8(base_e), EPW)],
                        dstage.at[pl.ds(0, EPW)])
        if companion:
            pltpu.sync_copy(src_hbm.at[pl.ds(_al8(base_e), EPW)],
                            sstage.at[pl.ds(0, EPW)])
        tv = dstage[pl.ds(4992, 16)]
        dstage[pl.ds(4992, 16)] = jnp.where(lanes >= 8, SENT, tv)
        if companion:
            tv2 = sstage[pl.ds(4992, 16)]
            sstage[pl.ds(4992, 16)] = jnp.where(lanes >= 8, 0, tv2)

        def cntf(g, carry):
            c0, c1 = carry
            dv = dstage[pl.ds(g * 16, 16)]
            bv = lax.shift_right_logical(dv * MAGIC, 24)
            for k in range(16):
                bk = bv[k]
                c0 = c0 + jnp.where(lanes == bk, 1, 0)
                c1 = c1 + jnp.where(lanes == (bk - 16), 1, 0)
            return (c0, c1)
        c0, c1 = lax.fori_loop(0, NV, cntf, (_zi16(), _zi16()))
        pc0 = jnp.bitwise_and(c0 + 7, -8)
        pc1 = jnp.bitwise_and(c1 + 7, -8)
        pub[pl.ds(0, 16)] = pc0
        pub[pl.ds(16, 16)] = pc1
        pltpu.sync_copy(pub, spm.at[pl.ds(_al8(s * 32), 32)])
        plsc.subcore_barrier()
        pltpu.sync_copy(spm, tbl)

        def offf(w, carry):
            o0, o1 = carry
            r0 = tbl[pl.ds(w * 32, 16)]
            r1 = tbl[pl.ds(w * 32 + 16, 16)]
            take = w < s
            o0 = o0 + jnp.where(take, r0, 0)
            o1 = o1 + jnp.where(take, r1, 0)
            return (o0, o1)
        g0, g1 = lax.fori_loop(0, 16, offf, (_zi16(), _zi16()))

        @pl.when(s == 0)
        def _():
            t0 = _zi16()
            t1 = _zi16()
            for w in range(16):
                t0 = t0 + tbl[pl.ds(w * 32, 16)]
                t1 = t1 + tbl[pl.ds(w * 32 + 16, 16)]
            for j in range(16):
                totv[pl.ds(j * 16, 16)] = jnp.broadcast_to(t0[j], (16,))
                totv[pl.ds(256 + j * 16, 16)] = jnp.broadcast_to(t1[j], (16,))
            pltpu.sync_copy(totv, tcnt_hbm.at[pl.ds(_al8(c * NB * 16), NB * 16)])

        lbs = []
        run = 0
        for b in range(32):
            lbs.append(run)
            run = run + (pc0[b] if b < 16 else pc1[b - 16])
        lbs.append(jnp.int32(DUMP0))
        lbs.append(jnp.int32(DUMP0))
        for j in range(17):
            posb[pl.ds(j * 16, 16)] = jnp.where(
                lanes < 8, lbs[2 * j], lbs[2 * j + 1])
        posb[pl.ds(272, 16)] = _zi16()

        sent16 = jnp.full((16,), SENT, _i32)
        zi = _zi16()

        def pre(i, carry):
            dloc[pl.ds(i * 16, 16)] = sent16
            if companion:
                sloc[pl.ds(i * 16, 16)] = zi
            return carry
        lax.fori_loop(0, LBUF // 16, pre, 0)

        one0 = jnp.where(lanes == 0, 1, 0)

        def place(g, carry):
            dv = dstage[pl.ds(g * 16, 16)]
            bv = lax.shift_right_logical(dv * MAGIC, 24)
            if companion:
                sv = sstage[pl.ds(g * 16, 16)]
            for k in range(16):
                bk = bv[k]
                po = _al8(bk * 8)
                pv = posb[pl.ds(po, 16)]
                pos = pv[0]
                posb[pl.ds(po, 16)] = pv + one0
                gp = jnp.bitwise_and(pos, -16)
                ln = pos - gp
                cd = dloc[pl.ds(gp, 16)]
                dloc[pl.ds(gp, 16)] = jnp.where(lanes == ln, dv[k], cd)
                if companion:
                    cs = sloc[pl.ds(gp, 16)]
                    sloc[pl.ds(gp, 16)] = jnp.where(lanes == ln, sv[k], cs)
            return carry
        lax.fori_loop(0, NV, place, 0)

        def flush(fire):
            for b in range(32):
                pcb = pc0[b] if b < 16 else pc1[b - 16]
                gob = g0[b] if b < 16 else g1[b - 16]
                gb = (c * NB + b) * CAP + gob
                ls = lbs[b]

                def run_copy(vloc, vhbm, off, size):
                    sref = vloc.at[pl.ds(_al8(ls + off), size)]
                    dref = vhbm.at[pl.ds(_al8(gb + off), size)]
                    if fire:
                        pltpu.async_copy(sref, dref, sem)
                    else:
                        pltpu.make_async_copy(sref, dref, sem).wait()

                q = pcb // 160

                def f1(i, carry):
                    run_copy(dloc, bdst_hbm, i * 160, 160)
                    if companion:
                        run_copy(sloc, bsrc_hbm, i * 160, 160)
                    return carry
                lax.fori_loop(0, q, f1, 0)
                r8 = (pcb - q * 160) // 8

                def f2(i, carry):
                    run_copy(dloc, bdst_hbm, q * 160 + i * 8, 8)
                    if companion:
                        run_copy(sloc, bsrc_hbm, q * 160 + i * 8, 8)
                    return carry
                lax.fori_loop(0, r8, f2, 0)
        flush(True)
        flush(False)

    return bucket


_bucket_pair = _make_bucket(True)
_bucket_val = _make_bucket(False)


@functools.partial(
    pl.kernel,
    out_type=jax.ShapeDtypeStruct((6 * NPAD,), _f32),
    mesh=_sc_mesh,
    scratch_types=(
        pltpu.VMEM((640,), _i32),
        pltpu.VMEM((336,), _f32),
        pltpu.VMEM((2 * NB * 16,), _i32),
    ),
)
def _degrees_kernel(l0, l1, l2, l3, l4, l5, t0, t1, t2, t3, t4, t5,
                    out_hbm, vbuf, hist, tcv):
    lists = (l0, l1, l2, l3, l4, l5)
    tcnts = (t0, t1, t2, t3, t4, t5)
    c = lax.axis_index("c")
    s = lax.axis_index("s")
    lanes = _lanes()
    b = c * 16 + s
    rbase = b * RPT

    for k in range(6):
        for i in range(336 // 16):
            hist[pl.ds(i * 16, 16)] = _z16()
        pltpu.sync_copy(tcnts[k], tcv)

        def upd16(off, valid):
            dv = vbuf[pl.ds(off, 16)]
            lv = dv - rbase
            ok = (lv >= 0) & (lv < RPT) & (lanes < valid)
            lv = jnp.where(ok, lv, RPT)
            for kk in range(16):
                li = lv[kk]
                gp = jnp.bitwise_and(li, -16)
                hv = hist[pl.ds(gp, 16)]
                hist[pl.ds(gp, 16)] = hv + jnp.where(lanes == (li - gp),
                                                     1.0, 0.0)

        for h2 in range(2):
            cnt = tcv[pl.ds(_al8((h2 * NB + b) * 16), 16)][0]
            gbase = (h2 * NB + b) * CAP
            q640 = cnt // 640

            def big(i, carry):
                pltpu.sync_copy(lists[k].at[pl.ds(_al8(gbase + i * 640), 640)],
                                vbuf)

                def grp(j, carry2):
                    upd16(j * 16, 16)
                    return carry2
                lax.fori_loop(0, 40, grp, 0)
                return carry
            lax.fori_loop(0, q640, big, 0)
            rem = cnt - q640 * 640
            q8 = rem // 8

            def small(i, carry):
                pltpu.sync_copy(
                    lists[k].at[pl.ds(_al8(gbase + q640 * 640 + i * 8), 8)],
                    vbuf.at[pl.ds(0, 8)])
                upd16(0, 8)
                return carry
            lax.fori_loop(0, q8, small, 0)

        pltpu.sync_copy(hist.at[pl.ds(0, RPT)],
                        out_hbm.at[pl.ds(_al8(k * NPAD + rbase), RPT)])


@functools.partial(
    pl.kernel,
    out_type=jax.ShapeDtypeStruct((NPAD, D), _f32),
    mesh=_sc_mesh,
    scratch_types=(
        pltpu.VMEM((RPT + 1, D), _f32),
        pltpu.VMEM((CH, D), _f32),
        pltpu.VMEM((CH,), _i32),
        pltpu.VMEM((CH,), _i32),
        pltpu.VMEM((2 * NB * 16,), _i32),
    ),
)
def _conv_kernel(h_hbm, bsrc, bdst, tcnt, out_hbm, acc, rows, sidx, dbuf, tcv):
    c = lax.axis_index("c")
    s = lax.axis_index("s")
    lanes = _lanes()
    b = c * 16 + s
    rbase = b * RPT

    def zr(r, carry):
        for j in range(D // 16):
            acc[r, pl.ds(j * 16, 16)] = _z16()
        return carry
    lax.fori_loop(0, RPT + 1, zr, 0)

    pltpu.sync_copy(tcnt, tcv)

    def addgrp(g, valid):
        dv = dbuf[pl.ds(g * 16, 16)]
        lv = dv - rbase
        ok = (lv >= 0) & (lv < RPT) & (lanes < valid)
        lv = jnp.where(ok, lv, RPT)
        for k in range(16):
            li = lv[k]
            e = g * 16 + k
            for j in range(D // 16):
                acc[li, pl.ds(j * 16, 16)] = (
                    acc[li, pl.ds(j * 16, 16)] + rows[e, pl.ds(j * 16, 16)])

    for h2 in range(2):
        cnt = tcv[pl.ds(_al8((h2 * NB + b) * 16), 16)][0]
        gbase = (h2 * NB + b) * CAP
        q = cnt // CH

        def chunk(i, carry):
            off = _al8(gbase + i * CH)
            pltpu.sync_copy(bsrc.at[pl.ds(off, CH)], sidx)
            pltpu.sync_copy(bdst.at[pl.ds(off, CH)], dbuf)
            pltpu.sync_copy(h_hbm.at[sidx], rows)

            def grp(g, carry2):
                addgrp(g, 16)
                return carry2
            lax.fori_loop(0, CH // 16, grp, 0)
            return carry
        lax.fori_loop(0, q, chunk, 0)
        r8 = (cnt - q * CH) // 8

        def tail(i, carry):
            off = _al8(gbase + q * CH + i * 8)
            pltpu.sync_copy(bsrc.at[pl.ds(off, 8)], sidx.at[pl.ds(0, 8)])
            pltpu.sync_copy(bdst.at[pl.ds(off, 8)], dbuf.at[pl.ds(0, 8)])
            pltpu.sync_copy(h_hbm.at[sidx.at[pl.ds(0, 8)]],
                            rows.at[pl.ds(0, 8)])
            addgrp(0, 8)
            return carry
        lax.fori_loop(0, r8, tail, 0)

    pltpu.sync_copy(acc.at[pl.ds(0, RPT)],
                    out_hbm.at[pl.ds(_al8(rbase), RPT)])


def _norms_body(deg_ref, out_ref):
    d = deg_ref[...]
    out_ref[...] = jnp.where(d > 0, lax.rsqrt(jnp.maximum(d, 1.0)),
                             jnp.zeros_like(d))


def _norms(deg):
    return pl.pallas_call(
        _norms_body,
        grid=(NPAD // 1024,),
        in_specs=[pl.BlockSpec((6, 1024), lambda i: (0, i))],
        out_specs=pl.BlockSpec((6, 1024), lambda i: (0, i)),
        out_shape=jax.ShapeDtypeStruct((6, NPAD), _f32),
    )(deg)


_BLK = 512
_GRID = NPAD // _BLK


def _row_spec():
    return pl.BlockSpec((_BLK, D), lambda i: (i, 0))


def _vec_spec():
    return pl.BlockSpec((_BLK, 1), lambda i: (i, 0))


def _prescale_body(x_ref, n1_ref, n2_ref, n3_ref, o1_ref, o2_ref, o3_ref):
    x = x_ref[...]
    o1_ref[...] = x * n1_ref[...]
    o2_ref[...] = x * n2_ref[...]
    o3_ref[...] = x * n3_ref[...]


def _prescale(x_pad, ns):
    shp = jax.ShapeDtypeStruct((NPAD, D), _f32)
    return pl.pallas_call(
        _prescale_body,
        grid=(_GRID,),
        in_specs=[_row_spec(), _vec_spec(), _vec_spec(), _vec_spec()],
        out_specs=[_row_spec()] * 3,
        out_shape=[shp] * 3,
    )(x_pad, *ns)


def _mid_body(a1, a2, a3, d1, d2, d3, s1, s2, s3,
              h1, h2, h3, hs1, hs2, hs3):
    for a, dn, sn, h, hs in ((a1, d1, s1, h1, hs1),
                             (a2, d2, s2, h2, hs2),
                             (a3, d3, s3, h3, hs3)):
        hv = a[...] * dn[...]
        h[...] = hv
        hs[...] = hv * sn[...]


def _mid(aggs, nds, nss):
    shp = jax.ShapeDtypeStruct((NPAD, D), _f32)
    return pl.pallas_call(
        _mid_body,
        grid=(_GRID,),
        in_specs=[_row_spec()] * 3 + [_vec_spec()] * 6,
        out_specs=[_row_spec()] * 6,
        out_shape=[shp] * 6,
    )(*aggs, *nds, *nss)


def _attn_body(l0, l1, l2, l3, a4, a5, a6, d1, d2, d3, w_ref, a_ref, out_ref):
    w = w_ref[...]
    av = a_ref[...]
    ts = [l0[...], l1[...], l2[...], l3[...],
          a4[...] * d1[...], a5[...] * d2[...], a6[...] * d3[...]]
    ms = [jnp.dot(t, w, preferred_element_type=_f32,
                  precision=lax.Precision.HIGHEST) * av for t in ts]
    mx = ms[0]
    for m in ms[1:]:
        mx = jnp.maximum(mx, m)
    es = [jnp.exp(m - mx) for m in ms]
    tot = es[0]
    for e in es[1:]:
        tot = tot + e
    acc = ts[0] * es[0]
    for t, e in zip(ts[1:], es[1:]):
        acc = acc + t * e
    out_ref[...] = acc / tot


def _attn(layers, nds, W, a):
    return pl.pallas_call(
        _attn_body,
        grid=(_GRID,),
        in_specs=([_row_spec()] * 7 + [_vec_spec()] * 3 +
                  [pl.BlockSpec((D, D), lambda i: (0, 0)),
                   pl.BlockSpec((1, D), lambda i: (0, 0))]),
        out_specs=_row_spec(),
        out_shape=jax.ShapeDtypeStruct((NPAD, D), _f32),
    )(*layers, *nds, W, a)


@jax.jit
def kernel(x, edge_index1, edge_index2, edge_index3, W_and, a_and):
    e1 = edge_index1.astype(_i32)
    e2 = edge_index2.astype(_i32)
    e3 = edge_index3.astype(_i32)
    srcs = [e1[0], e2[0], e3[0]]
    dsts = [e1[1], e2[1], e3[1]]

    x_pad = jnp.concatenate(
        [x.astype(_f32), jnp.zeros((NPAD - N, D), _f32)], axis=0)

    bsrc, bdst, tcnt = [], [], []
    for r in range(3):
        bs, bd, tc = _bucket_pair(srcs[r], dsts[r])
        bsrc.append(bs)
        bdst.append(bd)
        tcnt.append(tc)
    bval, vcnt = [], []
    for r in range(3):
        bv, vc = _bucket_val(srcs[r])
        bval.append(bv)
        vcnt.append(vc)

    deg_flat = _degrees_kernel(bdst[0], bdst[1], bdst[2],
                               bval[0], bval[1], bval[2],
                               tcnt[0], tcnt[1], tcnt[2],
                               vcnt[0], vcnt[1], vcnt[2])
    deg = deg_flat.reshape(6, NPAD)
    norms = _norms(deg)
    nd = [norms[r].reshape(NPAD, 1) for r in range(3)]
    ns = [norms[3 + r].reshape(NPAD, 1) for r in range(3)]

    xs = _prescale(x_pad, ns)

    a1 = [_conv_kernel(xs[r], bsrc[r], bdst[r], tcnt[r]) for r in range(3)]
    h11, h21, h31, hs11, hs21, hs31 = _mid(tuple(a1), nd, ns)
    hs = [hs11, hs21, hs31]
    a2 = [_conv_kernel(hs[r], bsrc[r], bdst[r], tcnt[r]) for r in range(3)]

    out_pad = _attn((x_pad, h11, h21, h31, a2[0], a2[1], a2[2]), nd,
                    W_and.astype(_f32), a_and.reshape(1, D).astype(_f32))
    return out_pad[:N]


# speedup vs baseline: 1.4122x; 1.1761x over previous
import functools

import jax
import jax.numpy as jnp
from jax import lax
from jax.experimental import pallas as pl
from jax.experimental.pallas import tpu as pltpu
from jax.experimental.pallas import tpu_sc as plsc

N = 10000
D = 256
E = 160000
NPAD = 10240
NB = 32
RPT = NPAD // NB
CAP = 80128
SUP = 2560
FLAT = 2 * NB * CAP + SUP
MAGIC = 52429
SENT = NPAD
EH = E // 2
EPW = EH // 16
NV = 5008 // 16
LBUF = 5280
DUMP0 = 5256
CH = 80

_f32 = jnp.float32
_i32 = jnp.int32

_sc_mesh = plsc.VectorSubcoreMesh(core_axis_name="c", subcore_axis_name="s")
_LANES = None


def _lanes():
    return lax.iota(_i32, 16)


def _al8(x):
    return pl.multiple_of(x, 8)


def _z16():
    return jnp.zeros((16,), _f32)


def _zi16():
    return jnp.zeros((16,), _i32)


def _make_bucket(companion):
    outs = [jax.ShapeDtypeStruct((FLAT,), _i32),
            jax.ShapeDtypeStruct((2 * NB * 16,), _i32)]
    scratch = [
        pltpu.VMEM((5008,), _i32),
        pltpu.VMEM((LBUF,), _i32),
        pltpu.VMEM((32,), _i32),
        pltpu.VMEM((512,), _i32),
        pltpu.VMEM((288,), _i32),
        pltpu.VMEM((2 * NB * 16 // 2,), _i32),
        pltpu.VMEM_SHARED((512,), _i32),
        pltpu.SemaphoreType.DMA,
    ]
    if companion:
        outs = [jax.ShapeDtypeStruct((FLAT,), _i32)] + outs
        scratch = [pltpu.VMEM((5008,), _i32),
                   pltpu.VMEM((LBUF,), _i32)] + scratch

    @functools.partial(pl.kernel, out_type=tuple(outs), mesh=_sc_mesh,
                       scratch_types=tuple(scratch))
    def bucket(*refs):
        if companion:
            (src_hbm, dst_hbm, bsrc_hbm, bdst_hbm, tcnt_hbm,
             sstage, sloc, dstage, dloc, pub, tbl, posb, totv, spm, sem) = refs
        else:
            (dst_hbm, bdst_hbm, tcnt_hbm,
             dstage, dloc, pub, tbl, posb, totv, spm, sem) = refs
        c = lax.axis_index("c")
        s = lax.axis_index("s")
        lanes = _lanes()
        base_e = c * EH + s * EPW

        pltpu.sync_copy(dst_hbm.at[pl.ds(_al8(base_e), EPW)],
                        dstage.at[pl.ds(0, EPW)])
        if companion:
            pltpu.sync_copy(src_hbm.at[pl.ds(_al8(base_e), EPW)],
                            sstage.at[pl.ds(0, EPW)])
        tv = dstage[pl.ds(4992, 16)]
        dstage[pl.ds(4992, 16)] = jnp.where(lanes >= 8, SENT, tv)
        if companion:
            tv2 = sstage[pl.ds(4992, 16)]
            sstage[pl.ds(4992, 16)] = jnp.where(lanes >= 8, 0, tv2)

        def cntf(g, carry):
            c0, c1 = carry
            dv = dstage[pl.ds(g * 16, 16)]
            bv = lax.shift_right_logical(dv * MAGIC, 24)
            for k in range(16):
                bk = bv[k]
                c0 = c0 + jnp.where(lanes == bk, 1, 0)
                c1 = c1 + jnp.where(lanes == (bk - 16), 1, 0)
            return (c0, c1)
        c0, c1 = lax.fori_loop(0, NV, cntf, (_zi16(), _zi16()))
        pc0 = jnp.bitwise_and(c0 + 7, -8)
        pc1 = jnp.bitwise_and(c1 + 7, -8)
        pub[pl.ds(0, 16)] = pc0
        pub[pl.ds(16, 16)] = pc1
        pltpu.sync_copy(pub, spm.at[pl.ds(_al8(s * 32), 32)])
        plsc.subcore_barrier()
        pltpu.sync_copy(spm, tbl)

        def offf(w, carry):
            o0, o1 = carry
            r0 = tbl[pl.ds(w * 32, 16)]
            r1 = tbl[pl.ds(w * 32 + 16, 16)]
            take = w < s
            o0 = o0 + jnp.where(take, r0, 0)
            o1 = o1 + jnp.where(take, r1, 0)
            return (o0, o1)
        g0, g1 = lax.fori_loop(0, 16, offf, (_zi16(), _zi16()))

        @pl.when(s == 0)
        def _():
            t0 = _zi16()
            t1 = _zi16()
            for w in range(16):
                t0 = t0 + tbl[pl.ds(w * 32, 16)]
                t1 = t1 + tbl[pl.ds(w * 32 + 16, 16)]
            for j in range(16):
                totv[pl.ds(j * 16, 16)] = jnp.broadcast_to(t0[j], (16,))
                totv[pl.ds(256 + j * 16, 16)] = jnp.broadcast_to(t1[j], (16,))
            pltpu.sync_copy(totv, tcnt_hbm.at[pl.ds(_al8(c * NB * 16), NB * 16)])

        lbs = []
        run = 0
        for b in range(32):
            lbs.append(run)
            run = run + (pc0[b] if b < 16 else pc1[b - 16])
        lbs.append(jnp.int32(DUMP0))
        lbs.append(jnp.int32(DUMP0))
        for j in range(17):
            posb[pl.ds(j * 16, 16)] = jnp.where(
                lanes < 8, lbs[2 * j], lbs[2 * j + 1])
        posb[pl.ds(272, 16)] = _zi16()

        sent16 = jnp.full((16,), SENT, _i32)
        zi = _zi16()

        def pre(i, carry):
            dloc[pl.ds(i * 16, 16)] = sent16
            if companion:
                sloc[pl.ds(i * 16, 16)] = zi
            return carry
        lax.fori_loop(0, LBUF // 16, pre, 0)

        one0 = jnp.where(lanes == 0, 1, 0)

        def place(g, carry):
            dv = dstage[pl.ds(g * 16, 16)]
            bv = lax.shift_right_logical(dv * MAGIC, 24)
            if companion:
                sv = sstage[pl.ds(g * 16, 16)]
            for k in range(16):
                bk = bv[k]
                po = _al8(bk * 8)
                pv = posb[pl.ds(po, 16)]
                pos = pv[0]
                posb[pl.ds(po, 16)] = pv + one0
                gp = jnp.bitwise_and(pos, -16)
                ln = pos - gp
                cd = dloc[pl.ds(gp, 16)]
                dloc[pl.ds(gp, 16)] = jnp.where(lanes == ln, dv[k], cd)
                if companion:
                    cs = sloc[pl.ds(gp, 16)]
                    sloc[pl.ds(gp, 16)] = jnp.where(lanes == ln, sv[k], cs)
            return carry
        lax.fori_loop(0, NV, place, 0)

        def flush(fire):
            for b in range(32):
                pcb = pc0[b] if b < 16 else pc1[b - 16]
                gob = g0[b] if b < 16 else g1[b - 16]
                gb = (c * NB + b) * CAP + gob
                ls = lbs[b]

                def run_copy(vloc, vhbm, off, size):
                    sref = vloc.at[pl.ds(_al8(ls + off), size)]
                    dref = vhbm.at[pl.ds(_al8(gb + off), size)]
                    if fire:
                        pltpu.async_copy(sref, dref, sem)
                    else:
                        pltpu.make_async_copy(sref, dref, sem).wait()

                q = pcb // 160

                def f1(i, carry):
                    run_copy(dloc, bdst_hbm, i * 160, 160)
                    if companion:
                        run_copy(sloc, bsrc_hbm, i * 160, 160)
                    return carry
                lax.fori_loop(0, q, f1, 0)
                r8 = (pcb - q * 160) // 8

                def f2(i, carry):
                    run_copy(dloc, bdst_hbm, q * 160 + i * 8, 8)
                    if companion:
                        run_copy(sloc, bsrc_hbm, q * 160 + i * 8, 8)
                    return carry
                lax.fori_loop(0, r8, f2, 0)
        flush(True)
        flush(False)

    return bucket


_bucket_pair = _make_bucket(True)
_bucket_val = _make_bucket(False)


@functools.partial(
    pl.kernel,
    out_type=jax.ShapeDtypeStruct((6 * NPAD,), _f32),
    mesh=_sc_mesh,
    scratch_types=(
        pltpu.VMEM((640,), _i32),
        pltpu.VMEM((336,), _f32),
        pltpu.VMEM((2 * NB * 16,), _i32),
    ),
)
def _degrees_kernel(l0, l1, l2, l3, l4, l5, t0, t1, t2, t3, t4, t5,
                    out_hbm, vbuf, hist, tcv):
    lists = (l0, l1, l2, l3, l4, l5)
    tcnts = (t0, t1, t2, t3, t4, t5)
    c = lax.axis_index("c")
    s = lax.axis_index("s")
    lanes = _lanes()
    b = c * 16 + s
    rbase = b * RPT

    for k in range(6):
        for i in range(336 // 16):
            hist[pl.ds(i * 16, 16)] = _z16()
        pltpu.sync_copy(tcnts[k], tcv)

        def upd16(off, valid):
            dv = vbuf[pl.ds(off, 16)]
            lv = dv - rbase
            ok = (lv >= 0) & (lv < RPT) & (lanes < valid)
            lv = jnp.where(ok, lv, RPT)
            for kk in range(16):
                li = lv[kk]
                gp = jnp.bitwise_and(li, -16)
                hv = hist[pl.ds(gp, 16)]
                hist[pl.ds(gp, 16)] = hv + jnp.where(lanes == (li - gp),
                                                     1.0, 0.0)

        for h2 in range(2):
            cnt = tcv[pl.ds(_al8((h2 * NB + b) * 16), 16)][0]
            gbase = (h2 * NB + b) * CAP
            q640 = cnt // 640

            def big(i, carry):
                pltpu.sync_copy(lists[k].at[pl.ds(_al8(gbase + i * 640), 640)],
                                vbuf)

                def grp(j, carry2):
                    upd16(j * 16, 16)
                    return carry2
                lax.fori_loop(0, 40, grp, 0)
                return carry
            lax.fori_loop(0, q640, big, 0)
            rem = cnt - q640 * 640
            q8 = rem // 8

            def small(i, carry):
                pltpu.sync_copy(
                    lists[k].at[pl.ds(_al8(gbase + q640 * 640 + i * 8), 8)],
                    vbuf.at[pl.ds(0, 8)])
                upd16(0, 8)
                return carry
            lax.fori_loop(0, q8, small, 0)

        pltpu.sync_copy(hist.at[pl.ds(0, RPT)],
                        out_hbm.at[pl.ds(_al8(k * NPAD + rbase), RPT)])


@functools.partial(
    pl.kernel,
    out_type=jax.ShapeDtypeStruct((NPAD, D), _f32),
    mesh=_sc_mesh,
    scratch_types=(
        pltpu.VMEM((RPT + 1, D), _f32),
        pltpu.VMEM((CH, D), _f32),
        pltpu.VMEM((CH, D), _f32),
        pltpu.VMEM((SUP,), _i32),
        pltpu.VMEM((SUP,), _i32),
        pltpu.VMEM((2 * NB * 16,), _i32),
        pltpu.SemaphoreType.DMA,
        pltpu.SemaphoreType.DMA,
    ),
)
def _conv_kernel(h_hbm, bsrc, bdst, tcnt, out_hbm, acc, rows_a, rows_b,
                 sidx, dbuf, tcv, sem_a, sem_b):
    c = lax.axis_index("c")
    s = lax.axis_index("s")
    lanes = _lanes()
    b = c * 16 + s
    rbase = b * RPT

    def zr(r, carry):
        for j in range(D // 16):
            acc[r, pl.ds(j * 16, 16)] = _z16()
        return carry
    lax.fori_loop(0, RPT + 1, zr, 0)

    pltpu.sync_copy(tcnt, tcv)

    def addgrp(rows, g, dvoff, valid):
        dv = dbuf[pl.ds(dvoff + g * 16, 16)]
        lv = dv - rbase
        ok = (lv >= 0) & (lv < RPT) & (lanes < valid)
        lv = jnp.where(ok, lv, RPT)
        for k in range(16):
            li = lv[k]
            e = g * 16 + k
            for j in range(D // 16):
                acc[li, pl.ds(j * 16, 16)] = (
                    acc[li, pl.ds(j * 16, 16)] + rows[e, pl.ds(j * 16, 16)])

    SPC = SUP // CH

    def fire(j, rows, sem):
        pltpu.async_copy(h_hbm.at[sidx.at[pl.ds(_al8(j * CH), CH)]],
                         rows, sem)

    def wait(j, rows, sem):
        pltpu.make_async_copy(h_hbm.at[sidx.at[pl.ds(_al8(j * CH), CH)]],
                              rows, sem).wait()

    for h2 in range(2):
        cnt = tcv[pl.ds(_al8((h2 * NB + b) * 16), 16)][0]
        gbase = (h2 * NB + b) * CAP
        q = cnt // CH
        nsup = (q * CH + SUP - 1) // SUP

        def sup_body(sup, carry):
            soff = _al8(gbase + sup * SUP)
            pltpu.sync_copy(bsrc.at[pl.ds(soff, SUP)], sidx)
            pltpu.sync_copy(bdst.at[pl.ds(soff, SUP)], dbuf)
            ncl = jnp.minimum(q - sup * SPC, SPC)
            fire(0, rows_a, sem_a)

            def chunk(j, carry2):
                p = jnp.bitwise_and(j, 1)

                @pl.when((p == 0) & (j + 1 < ncl))
                def _():
                    fire(j + 1, rows_b, sem_b)

                @pl.when((p == 1) & (j + 1 < ncl))
                def _():
                    fire(j + 1, rows_a, sem_a)

                @pl.when(p == 0)
                def _():
                    wait(j, rows_a, sem_a)

                    def grp(g, carry3):
                        addgrp(rows_a, g, j * CH, 16)
                        return carry3
                    lax.fori_loop(0, CH // 16, grp, 0)

                @pl.when(p == 1)
                def _():
                    wait(j, rows_b, sem_b)

                    def grp(g, carry3):
                        addgrp(rows_b, g, j * CH, 16)
                        return carry3
                    lax.fori_loop(0, CH // 16, grp, 0)
                return carry2
            lax.fori_loop(0, ncl, chunk, 0)
            return carry
        lax.fori_loop(0, nsup, sup_body, 0)

        r8 = (cnt - q * CH) // 8

        def tail(i, carry):
            off = _al8(gbase + q * CH + i * 8)
            pltpu.sync_copy(bsrc.at[pl.ds(off, 8)], sidx.at[pl.ds(0, 8)])
            pltpu.sync_copy(bdst.at[pl.ds(off, 8)], dbuf.at[pl.ds(0, 8)])
            pltpu.sync_copy(h_hbm.at[sidx.at[pl.ds(0, 8)]],
                            rows_a.at[pl.ds(0, 8)])
            addgrp(rows_a, 0, 0, 8)
            return carry
        lax.fori_loop(0, r8, tail, 0)

    pltpu.sync_copy(acc.at[pl.ds(0, RPT)],
                    out_hbm.at[pl.ds(_al8(rbase), RPT)])


def _norms_body(deg_ref, out_ref):
    d = deg_ref[...]
    out_ref[...] = jnp.where(d > 0, lax.rsqrt(jnp.maximum(d, 1.0)),
                             jnp.zeros_like(d))


def _norms(deg):
    return pl.pallas_call(
        _norms_body,
        grid=(NPAD // 1024,),
        in_specs=[pl.BlockSpec((6, 1024), lambda i: (0, i))],
        out_specs=pl.BlockSpec((6, 1024), lambda i: (0, i)),
        out_shape=jax.ShapeDtypeStruct((6, NPAD), _f32),
    )(deg)


_BLK = 512
_GRID = NPAD // _BLK


def _row_spec():
    return pl.BlockSpec((_BLK, D), lambda i: (i, 0))


def _vec_spec():
    return pl.BlockSpec((_BLK, 1), lambda i: (i, 0))


def _prescale_body(x_ref, n1_ref, n2_ref, n3_ref, o1_ref, o2_ref, o3_ref):
    x = x_ref[...]
    o1_ref[...] = x * n1_ref[...]
    o2_ref[...] = x * n2_ref[...]
    o3_ref[...] = x * n3_ref[...]


def _prescale(x_pad, ns):
    shp = jax.ShapeDtypeStruct((NPAD, D), _f32)
    return pl.pallas_call(
        _prescale_body,
        grid=(_GRID,),
        in_specs=[_row_spec(), _vec_spec(), _vec_spec(), _vec_spec()],
        out_specs=[_row_spec()] * 3,
        out_shape=[shp] * 3,
    )(x_pad, *ns)


def _mid_body(a1, a2, a3, d1, d2, d3, s1, s2, s3,
              h1, h2, h3, hs1, hs2, hs3):
    for a, dn, sn, h, hs in ((a1, d1, s1, h1, hs1),
                             (a2, d2, s2, h2, hs2),
                             (a3, d3, s3, h3, hs3)):
        hv = a[...] * dn[...]
        h[...] = hv
        hs[...] = hv * sn[...]


def _mid(aggs, nds, nss):
    shp = jax.ShapeDtypeStruct((NPAD, D), _f32)
    return pl.pallas_call(
        _mid_body,
        grid=(_GRID,),
        in_specs=[_row_spec()] * 3 + [_vec_spec()] * 6,
        out_specs=[_row_spec()] * 6,
        out_shape=[shp] * 6,
    )(*aggs, *nds, *nss)


def _attn_body(l0, l1, l2, l3, a4, a5, a6, d1, d2, d3, w_ref, a_ref, out_ref):
    w = w_ref[...]
    av = a_ref[...]
    ts = [l0[...], l1[...], l2[...], l3[...],
          a4[...] * d1[...], a5[...] * d2[...], a6[...] * d3[...]]
    ms = [jnp.dot(t, w, preferred_element_type=_f32,
                  precision=lax.Precision.HIGHEST) * av for t in ts]
    mx = ms[0]
    for m in ms[1:]:
        mx = jnp.maximum(mx, m)
    es = [jnp.exp(m - mx) for m in ms]
    tot = es[0]
    for e in es[1:]:
        tot = tot + e
    acc = ts[0] * es[0]
    for t, e in zip(ts[1:], es[1:]):
        acc = acc + t * e
    out_ref[...] = acc / tot


def _attn(layers, nds, W, a):
    return pl.pallas_call(
        _attn_body,
        grid=(_GRID,),
        in_specs=([_row_spec()] * 7 + [_vec_spec()] * 3 +
                  [pl.BlockSpec((D, D), lambda i: (0, 0)),
                   pl.BlockSpec((1, D), lambda i: (0, 0))]),
        out_specs=_row_spec(),
        out_shape=jax.ShapeDtypeStruct((NPAD, D), _f32),
    )(*layers, *nds, W, a)


@jax.jit
def kernel(x, edge_index1, edge_index2, edge_index3, W_and, a_and):
    e1 = edge_index1.astype(_i32)
    e2 = edge_index2.astype(_i32)
    e3 = edge_index3.astype(_i32)
    srcs = [e1[0], e2[0], e3[0]]
    dsts = [e1[1], e2[1], e3[1]]

    x_pad = jnp.concatenate(
        [x.astype(_f32), jnp.zeros((NPAD - N, D), _f32)], axis=0)

    bsrc, bdst, tcnt = [], [], []
    for r in range(3):
        bs, bd, tc = _bucket_pair(srcs[r], dsts[r])
        bsrc.append(bs)
        bdst.append(bd)
        tcnt.append(tc)
    bval, vcnt = [], []
    for r in range(3):
        bv, vc = _bucket_val(srcs[r])
        bval.append(bv)
        vcnt.append(vc)

    deg_flat = _degrees_kernel(bdst[0], bdst[1], bdst[2],
                               bval[0], bval[1], bval[2],
                               tcnt[0], tcnt[1], tcnt[2],
                               vcnt[0], vcnt[1], vcnt[2])
    deg = deg_flat.reshape(6, NPAD)
    norms = _norms(deg)
    nd = [norms[r].reshape(NPAD, 1) for r in range(3)]
    ns = [norms[3 + r].reshape(NPAD, 1) for r in range(3)]

    xs = _prescale(x_pad, ns)

    a1 = [_conv_kernel(xs[r], bsrc[r], bdst[r], tcnt[r]) for r in range(3)]
    h11, h21, h31, hs11, hs21, hs31 = _mid(tuple(a1), nd, ns)
    hs = [hs11, hs21, hs31]
    a2 = [_conv_kernel(hs[r], bsrc[r], bdst[r], tcnt[r]) for r in range(3)]

    out_pad = _attn((x_pad, h11, h21, h31, a2[0], a2[1], a2[2]), nd,
                    W_and.astype(_f32), a_and.reshape(1, D).astype(_f32))
    return out_pad[:N]
